# Initial kernel scaffold; baseline (speedup 1.0000x reference)
import jax
import jax.numpy as jnp
from jax.experimental import pallas as pl


def kernel(z_what, z_where, z_present, z_depth, W1, b1, W2, b2, W3, b3):
    raise NotImplementedError("write your pallas kernel here")


# speedup vs baseline: 2444.7821x; 2444.7821x over previous
import jax
import jax.numpy as jnp
from jax.experimental import pallas as pl
from jax.experimental.pallas import tpu as pltpu

ZW = 64
H1 = 256
H2 = 1024
S = 64
IMG = 128
OUT = 3 * S * S
COLT = 1024
NT = OUT // COLT


def _mlp_body(z_ref, w1_ref, b1_ref, w2_ref, b2_ref, w3_ref, b3_ref,
              out_ref, h2_ref):
    t = pl.program_id(0)

    @pl.when(t == 0)
    def _():
        h1 = jax.nn.relu(
            jnp.dot(z_ref[...], w1_ref[...], preferred_element_type=jnp.float32)
            + b1_ref[...])
        h2_ref[...] = jax.nn.relu(
            jnp.dot(h1, w2_ref[...], preferred_element_type=jnp.float32)
            + b2_ref[...])

    o = jnp.dot(h2_ref[...], w3_ref[...], preferred_element_type=jnp.float32)
    out_ref[...] = jax.nn.sigmoid(o + b3_ref[...])


def _stn_body(nobj, n, dec_ref, zw_ref, d_ref, p_ref, out_ref, acc_ref):
    i = pl.program_id(1)
    b = pl.program_id(0)
    row = b * nobj + i

    a = dec_ref[0]

    cx = zw_ref[row, 0] * 2.0 - 1.0
    cy = zw_ref[row, 1] * 2.0 - 1.0
    ww = jnp.maximum(zw_ref[row, 2], 1e-2)
    hh = jnp.maximum(zw_ref[row, 3], 1e-2)

    q = jax.lax.broadcasted_iota(jnp.int32, (1, IMG), 1).astype(jnp.float32)
    gx = (q + 0.5) / (IMG / 2.0) - 1.0
    u = ((gx - cx) / ww + 1.0) * (S / 2.0) - 0.5
    u0 = jnp.floor(u)
    du = u - u0
    l = jax.lax.broadcasted_iota(jnp.int32, (IMG, 1), 0).astype(jnp.float32)

    def rx(xsrc, lane_mask):
        c0 = jnp.where((xsrc == u0) & (u0 >= 0.0) & (u0 <= S - 1.0),
                       1.0 - du, 0.0)
        c1 = jnp.where((xsrc == u0 + 1.0) & (u0 + 1.0 >= 0.0)
                       & (u0 + 1.0 <= S - 1.0), du, 0.0)
        return jnp.where(lane_mask, c0 + c1, 0.0)

    rx0 = rx(l, l < S)
    rx1 = rx(l - S, l >= S)

    pp = jax.lax.broadcasted_iota(jnp.int32, (3 * IMG, 1), 0)
    orow = (pp % IMG).astype(jnp.float32)
    gy = (orow + 0.5) / (IMG / 2.0) - 1.0
    v = ((gy - cy) / hh + 1.0) * (S / 2.0) - 0.5
    v0 = jnp.floor(v)
    dv = v - v0
    rr = jax.lax.broadcasted_iota(jnp.int32, (1, 96), 1)
    cmatch = (pp // IMG) == (rr // 32)
    ybase = (2 * (rr % 32)).astype(jnp.float32)

    def ry(ysrc):
        c0 = jnp.where((ysrc == v0) & (v0 >= 0.0) & (v0 <= S - 1.0),
                       1.0 - dv, 0.0)
        c1 = jnp.where((ysrc == v0 + 1.0) & (v0 + 1.0 >= 0.0)
                       & (v0 + 1.0 <= S - 1.0), dv, 0.0)
        return jnp.where(cmatch, c0 + c1, 0.0)

    ry0 = ry(ybase)
    ry1 = ry(ybase + 1.0)

    canvas = (jnp.dot(ry0, jnp.dot(a, rx0, preferred_element_type=jnp.float32),
                      preferred_element_type=jnp.float32)
              + jnp.dot(ry1, jnp.dot(a, rx1, preferred_element_type=jnp.float32),
                        preferred_element_type=jnp.float32))

    dvec = d_ref[0]
    pvec = p_ref[0]
    deff = jnp.where(pvec == 1.0, dvec, -1e30)
    e = jnp.exp(deff - jnp.max(deff))
    wv = e / jnp.sum(e)
    sel = jax.lax.broadcasted_iota(jnp.int32, (1, n), 1) == i
    wgt = jnp.sum(jnp.where(sel, wv, 0.0))

    contrib = wgt * canvas

    @pl.when(i == 0)
    def _():
        acc_ref[...] = contrib

    @pl.when(jnp.logical_and(i > 0, i < nobj - 1))
    def _():
        acc_ref[...] += contrib

    @pl.when(i == nobj - 1)
    def _():
        merged = acc_ref[...]
        mask = jnp.where(merged < 0.001, 1.0, 0.0)
        out_ref[0] = merged + canvas * mask


def kernel(z_what, z_where, z_present, z_depth, W1, b1, W2, b2, W3, b3):
    B, nobj, _ = z_what.shape
    n = nobj - 1
    M = B * nobj

    z = z_what.reshape(M, ZW)
    bg = jnp.broadcast_to(jnp.array([0.5, 0.5, 1.0, 1.0], jnp.float32),
                          (B, 1, 4))
    zw = jnp.concatenate([z_where, bg], axis=1).reshape(M, 4)
    d = z_depth.reshape(B, 1, n)
    p = z_present.reshape(B, 1, n)

    decoded = pl.pallas_call(
        _mlp_body,
        grid=(NT,),
        in_specs=[
            pl.BlockSpec((M, ZW), lambda t: (0, 0)),
            pl.BlockSpec((ZW, H1), lambda t: (0, 0)),
            pl.BlockSpec((1, H1), lambda t: (0, 0)),
            pl.BlockSpec((H1, H2), lambda t: (0, 0)),
            pl.BlockSpec((1, H2), lambda t: (0, 0)),
            pl.BlockSpec((H2, COLT), lambda t: (0, t)),
            pl.BlockSpec((1, COLT), lambda t: (0, t)),
        ],
        out_specs=pl.BlockSpec((M, COLT), lambda t: (0, t)),
        out_shape=jax.ShapeDtypeStruct((M, OUT), jnp.float32),
        scratch_shapes=[pltpu.VMEM((M, H2), jnp.float32)],
        compiler_params=pltpu.CompilerParams(
            dimension_semantics=("arbitrary",)),
    )(z, W1, b1.reshape(1, H1), W2, b2.reshape(1, H2), W3,
      b3.reshape(1, OUT))

    dec3 = decoded.reshape(M, OUT // 128, 128)

    import functools
    body = functools.partial(_stn_body, nobj, n)
    out = pl.pallas_call(
        body,
        grid=(B, nobj),
        in_specs=[
            pl.BlockSpec((1, OUT // 128, 128), lambda b, i: (b * nobj + i, 0, 0)),
            pl.BlockSpec(memory_space=pltpu.SMEM),
            pl.BlockSpec((1, 1, n), lambda b, i: (b, 0, 0)),
            pl.BlockSpec((1, 1, n), lambda b, i: (b, 0, 0)),
        ],
        out_specs=pl.BlockSpec((1, 3 * IMG, IMG), lambda b, i: (b, 0, 0)),
        out_shape=jax.ShapeDtypeStruct((B, 3 * IMG, IMG), jnp.float32),
        scratch_shapes=[pltpu.VMEM((3 * IMG, IMG), jnp.float32)],
        compiler_params=pltpu.CompilerParams(
            dimension_semantics=("arbitrary", "arbitrary")),
    )(dec3, zw, d, p)

    return out.reshape(B, 3, IMG, IMG)


# speedup vs baseline: 2678.9430x; 1.0958x over previous
import jax
import jax.numpy as jnp
from jax.experimental import pallas as pl
from jax.experimental.pallas import tpu as pltpu

ZW = 64
H1 = 256
H2 = 1024
S = 64
IMG = 128
OUT = 3 * S * S
COLT = 1024
NT = OUT // COLT


def _mlp_body(z_ref, w1_ref, b1_ref, w2_ref, b2_ref, w3_ref, b3_ref,
              out_ref, h2_ref):
    t = pl.program_id(0)

    @pl.when(t == 0)
    def _():
        h1 = jax.nn.relu(
            jnp.dot(z_ref[...], w1_ref[...], preferred_element_type=jnp.float32)
            + b1_ref[...])
        h2_ref[...] = jax.nn.relu(
            jnp.dot(h1, w2_ref[...], preferred_element_type=jnp.float32)
            + b2_ref[...])

    o = jnp.dot(h2_ref[...], w3_ref[...], preferred_element_type=jnp.float32)
    out_ref[...] = jax.nn.sigmoid(o + b3_ref[...])


def _stn_body(nobj, n, dec_ref, zw_ref, d_ref, p_ref, out_ref, acc_ref):
    i = pl.program_id(1)
    b = pl.program_id(0)
    row = b * nobj + i

    a = dec_ref[0]

    cx = zw_ref[row, 0] * 2.0 - 1.0
    cy = zw_ref[row, 1] * 2.0 - 1.0
    ww = jnp.maximum(zw_ref[row, 2], 1e-2)
    hh = jnp.maximum(zw_ref[row, 3], 1e-2)

    q = jax.lax.broadcasted_iota(jnp.int32, (1, IMG), 1).astype(jnp.float32)
    gx = (q + 0.5) / (IMG / 2.0) - 1.0
    u = ((gx - cx) / ww + 1.0) * (S / 2.0) - 0.5
    u0 = jnp.floor(u)
    du = u - u0
    t0 = jnp.where((u0 >= 0.0) & (u0 <= S - 1.0), 1.0 - du, 0.0)
    t1 = jnp.where((u0 + 1.0 >= 0.0) & (u0 + 1.0 <= S - 1.0), du, 0.0)
    l = jax.lax.broadcasted_iota(jnp.int32, (IMG, 1), 0).astype(jnp.float32)

    def rx(xsrc, lane_mask):
        m0 = jnp.where((xsrc == u0) & lane_mask, t0, 0.0)
        m1 = jnp.where((xsrc == u0 + 1.0) & lane_mask, t1, 0.0)
        return m0 + m1

    rx0 = rx(l, l < S)
    rx1 = rx(l - S, l >= S)

    b0 = jnp.dot(a, rx0, preferred_element_type=jnp.float32)
    b1 = jnp.dot(a, rx1, preferred_element_type=jnp.float32)

    pp = jax.lax.broadcasted_iota(jnp.int32, (IMG, 1), 0).astype(jnp.float32)
    gy = (pp + 0.5) / (IMG / 2.0) - 1.0
    v = ((gy - cy) / hh + 1.0) * (S / 2.0) - 0.5
    v0 = jnp.floor(v)
    dv = v - v0
    s0 = jnp.where((v0 >= 0.0) & (v0 <= S - 1.0), 1.0 - dv, 0.0)
    s1 = jnp.where((v0 + 1.0 >= 0.0) & (v0 + 1.0 <= S - 1.0), dv, 0.0)
    jj = jax.lax.broadcasted_iota(jnp.int32, (1, 32), 1).astype(jnp.float32)

    def ry_core(ysrc):
        m0 = jnp.where(ysrc == v0, s0, 0.0)
        m1 = jnp.where(ysrc == v0 + 1.0, s1, 0.0)
        return m0 + m1

    c0 = ry_core(2.0 * jj)
    c1 = ry_core(2.0 * jj + 1.0)

    canvas = jnp.concatenate(
        [jnp.dot(c0, b0[32 * c:32 * (c + 1), :],
                 preferred_element_type=jnp.float32)
         + jnp.dot(c1, b1[32 * c:32 * (c + 1), :],
                   preferred_element_type=jnp.float32)
         for c in range(3)], axis=0)

    dvec = d_ref[0]
    pvec = p_ref[0]
    deff = jnp.where(pvec == 1.0, dvec, -1e30)
    e = jnp.exp(deff - jnp.max(deff))
    wv = e / jnp.sum(e)
    sel = jax.lax.broadcasted_iota(jnp.int32, (1, n), 1) == i
    wgt = jnp.sum(jnp.where(sel, wv, 0.0))

    contrib = wgt * canvas

    @pl.when(i == 0)
    def _():
        acc_ref[...] = contrib

    @pl.when(jnp.logical_and(i > 0, i < nobj - 1))
    def _():
        acc_ref[...] += contrib

    @pl.when(i == nobj - 1)
    def _():
        merged = acc_ref[...]
        mask = jnp.where(merged < 0.001, 1.0, 0.0)
        out_ref[0] = merged + canvas * mask


def kernel(z_what, z_where, z_present, z_depth, W1, b1, W2, b2, W3, b3):
    B, nobj, _ = z_what.shape
    n = nobj - 1
    M = B * nobj

    z = z_what.reshape(M, ZW)
    bg = jnp.broadcast_to(jnp.array([0.5, 0.5, 1.0, 1.0], jnp.float32),
                          (B, 1, 4))
    zw = jnp.concatenate([z_where, bg], axis=1).reshape(M, 4)
    d = z_depth.reshape(B, 1, n)
    p = z_present.reshape(B, 1, n)

    decoded = pl.pallas_call(
        _mlp_body,
        grid=(NT,),
        in_specs=[
            pl.BlockSpec((M, ZW), lambda t: (0, 0)),
            pl.BlockSpec((ZW, H1), lambda t: (0, 0)),
            pl.BlockSpec((1, H1), lambda t: (0, 0)),
            pl.BlockSpec((H1, H2), lambda t: (0, 0)),
            pl.BlockSpec((1, H2), lambda t: (0, 0)),
            pl.BlockSpec((H2, COLT), lambda t: (0, t)),
            pl.BlockSpec((1, COLT), lambda t: (0, t)),
        ],
        out_specs=pl.BlockSpec((M, COLT), lambda t: (0, t)),
        out_shape=jax.ShapeDtypeStruct((M, OUT), jnp.float32),
        scratch_shapes=[pltpu.VMEM((M, H2), jnp.float32)],
        compiler_params=pltpu.CompilerParams(
            dimension_semantics=("arbitrary",)),
    )(z, W1, b1.reshape(1, H1), W2, b2.reshape(1, H2), W3,
      b3.reshape(1, OUT))

    dec3 = decoded.reshape(M, OUT // 128, 128)

    import functools
    body = functools.partial(_stn_body, nobj, n)
    out = pl.pallas_call(
        body,
        grid=(B, nobj),
        in_specs=[
            pl.BlockSpec((1, OUT // 128, 128), lambda b, i: (b * nobj + i, 0, 0)),
            pl.BlockSpec(memory_space=pltpu.SMEM),
            pl.BlockSpec((1, 1, n), lambda b, i: (b, 0, 0)),
            pl.BlockSpec((1, 1, n), lambda b, i: (b, 0, 0)),
        ],
        out_specs=pl.BlockSpec((1, 3 * IMG, IMG), lambda b, i: (b, 0, 0)),
        out_shape=jax.ShapeDtypeStruct((B, 3 * IMG, IMG), jnp.float32),
        scratch_shapes=[pltpu.VMEM((3 * IMG, IMG), jnp.float32)],
        compiler_params=pltpu.CompilerParams(
            dimension_semantics=("arbitrary", "arbitrary")),
    )(dec3, zw, d, p)

    return out.reshape(B, 3, IMG, IMG)
